# baseline (speedup 1.0000x reference)
import functools

import jax
import jax.numpy as jnp
from jax import lax
from jax.experimental import pallas as pl
from jax.experimental.pallas import tpu as pltpu
from jax.experimental.pallas import tpu_sc as plsc

_NUM_EMBEDDINGS = 1000000
_DIM = 32
_BATCH = 16384
_PACK = 128 // _DIM

_info = plsc.get_sparse_core_info()
_NC, _NS, _L = _info.num_cores, _info.num_subcores, _info.num_lanes
_NW = _NC * _NS
_BPW = _BATCH // _NW
_C = 128
_NCH = _BPW // _C

_mesh = plsc.VectorSubcoreMesh(core_axis_name="c", subcore_axis_name="s")


@functools.partial(
    pl.kernel,
    mesh=_mesh,
    out_type=jax.ShapeDtypeStruct((_BATCH, _DIM), jnp.float32),
    scratch_types=[
        pltpu.VMEM((_BPW,), jnp.int32),
        pltpu.VMEM((_BPW,), jnp.int32),
        pltpu.VMEM((_BPW // 2, 128), jnp.float32),
        pltpu.VMEM((_BPW, _DIM), jnp.float32),
        pltpu.SemaphoreType.DMA,
    ],
    compiler_params=pltpu.CompilerParams(needs_layout_passes=False),
)
def _gather_kernel(idx_hbm, rtab_hbm, out_hbm, idx_v, gidx_v, big_v,
                   out_v, sem):
    wid = lax.axis_index("s") * _NC + lax.axis_index("c")
    base = wid * _BPW
    pltpu.sync_copy(idx_hbm.at[pl.ds(base, _BPW)], idx_v)

    for k in range(_BPW // _L):
        v = idx_v[pl.ds(k * _L, _L)]
        gidx_v[pl.ds(k * _L, _L)] = lax.shift_right_logical(v, 2)

    lane = lax.iota(jnp.int32, _L)
    half = _BPW // 2

    for h in range(2):
        copies = []
        for ch in range(half // _C):
            copies.append(
                pltpu.async_copy(
                    rtab_hbm.at[gidx_v.at[pl.ds(h * half + ch * _C, _C)]],
                    big_v.at[pl.ds(ch * _C, _C)],
                    sem,
                )
            )
        for cp in copies:
            cp.wait()

        def body(g, carry):
            rowid = lane + g * _L
            cbase = lax.shift_left(
                jnp.bitwise_and(idx_v[pl.ds(h * half + g * _L, _L)],
                                _PACK - 1), 5)
            for j in range(_DIM):
                vals = plsc.load_gather(big_v, [rowid, cbase + j])
                plsc.store_scatter(out_v, [rowid + h * half, lane * 0 + j],
                                   vals)
            return carry

        lax.fori_loop(0, half // _L, body, 0)

    pltpu.sync_copy(out_v, out_hbm.at[pl.ds(base, _BPW)])


def kernel(input_index, embeds):
    idx = input_index.astype(jnp.int32)
    rtab = embeds.reshape(_NUM_EMBEDDINGS // _PACK, _DIM * _PACK)
    return _gather_kernel(idx, rtab)
